# baseline (speedup 1.0000x reference)
import jax
import jax.numpy as jnp
import numpy as np
from jax import lax
from jax.experimental import pallas as pl
from jax.experimental.pallas import tpu as pltpu
from jax.experimental.pallas import tpu_sc as plsc

BATCH = 16384
WIDTH = 128
NROWS_PAD = 1320
ROWS = BATCH * 4
NC, NS = 2, 16
NW = NC * NS
RPW = ROWS // NW
CH = 128
NCH = RPW // CH

_QSRC = (0, 2, 3, 1)


def _static_m():
    sig = np.zeros((4, 9), dtype=np.int64)
    for p in range(9):
        x, y = divmod(p, 3)
        sig[0, p] = p
        sig[1, p] = 3 * (2 - y) + x
        sig[2, p] = 8 - p
        sig[3, p] = 3 * y + (2 - x)
    inv = np.zeros_like(sig)
    for j in range(4):
        inv[j, sig[j]] = np.arange(9)

    m = np.zeros((NROWS_PAD, 46), np.float32)
    for j in range(4):
        base = 324 * j
        for ia in range(81):
            v = ia
            for t in range(4):
                d = v % 3
                v //= 3
                m[base + ia, 3 * inv[j][t] + d] += 1
        for ib in range(243):
            v = ib
            for t in range(5):
                d = v % 3
                v //= 3
                m[base + 81 + ib, 3 * inv[j][4 + t] + d] += 1
    for c in range(19):
        m[1296 + c, 27 + c] = 1
    return m


_M = _static_m()


def _table_body(m_ref, w_ref, t_ref):
    t_ref[...] = jnp.dot(m_ref[...], w_ref[...],
                         preferred_element_type=jnp.float32)


def _idx_body(b3_ref, e_ref, oa_ref, ob_ref, oc_ref):
    b3 = b3_ref[...]
    cnt = (b3[:, :, 0] != 0).astype(jnp.int32)
    for e in range(1, 36):
        cnt = cnt + (b3[:, :, e] != 0)
    ias = []
    ibs = []
    for j in range(4):
        q9 = 9 * _QSRC[j]
        ia = b3[:, :, q9]
        for t in range(1, 4):
            ia = ia + (3 ** t) * b3[:, :, q9 + t]
        ib = b3[:, :, q9 + 4]
        for t in range(1, 5):
            ib = ib + (3 ** t) * b3[:, :, q9 + 4 + t]
        ias.append(ia.astype(jnp.float32))
        ibs.append(ib.astype(jnp.float32))
    e_m = e_ref[...]
    rblk = b3.shape[0]
    jlane = lax.broadcasted_iota(jnp.int32, (rblk, 128), 1) % 4
    sa = jnp.dot(jnp.concatenate(ias, axis=1), e_m,
                 preferred_element_type=jnp.float32).astype(jnp.int32)
    sb = jnp.dot(jnp.concatenate(ibs, axis=1), e_m,
                 preferred_element_type=jnp.float32).astype(jnp.int32)
    sc = jnp.dot(jnp.concatenate([cnt.astype(jnp.float32)] * 4, axis=1), e_m,
                 preferred_element_type=jnp.float32).astype(jnp.int32)
    oa_ref[...] = sa + 324 * jlane
    ob_ref[...] = sb + 324 * jlane + 81
    oc_ref[...] = jnp.where(sc < 19, 1296 + sc, 1315)


def _sc_body(t_hbm, ia_hbm, ib_hbm, ic_hbm, out_hbm,
             ia_v, ib_v, ic_v, acc_v, t_sh,
             sem_i, sem_a, sem_bc, sem_o):
    cid = lax.axis_index("c")
    sid = lax.axis_index("s")
    wid = sid * NC + cid
    base = wid * RPW

    @pl.when(sid == 0)
    def _():
        pltpu.sync_copy(t_hbm, t_sh)

    plsc.subcore_barrier()

    def issue_i(ch):
        r0 = base + ch * CH
        return (
            pltpu.async_copy(ia_hbm.at[pl.ds(r0, CH)], ia_v.at[ch & 1], sem_i),
            pltpu.async_copy(ib_hbm.at[pl.ds(r0, CH)], ib_v.at[ch & 1], sem_i),
            pltpu.async_copy(ic_hbm.at[pl.ds(r0, CH)], ic_v.at[ch & 1], sem_i),
        )

    def issue_a(ch):
        return pltpu.async_copy(t_sh.at[ia_v.at[ch & 1]],
                                acc_v.at[ch % 3], sem_a)

    def issue_bc(ch):
        return (
            pltpu.async_copy(t_sh.at[ib_v.at[ch & 1]],
                             acc_v.at[ch % 3], sem_bc, add=True),
            pltpu.async_copy(t_sh.at[ic_v.at[ch & 1]],
                             acc_v.at[ch % 3], sem_bc, add=True),
        )

    def issue_o(ch):
        return pltpu.async_copy(acc_v.at[ch % 3],
                                out_hbm.at[pl.ds(base + ch * CH, CH)], sem_o)

    di = {0: issue_i(0)}
    for c in di.pop(0):
        c.wait()
    da = {0: issue_a(0)}
    dbc = {}
    do = {}
    for ch in range(NCH):
        if ch + 1 < NCH:
            di[ch + 1] = issue_i(ch + 1)
        da.pop(ch).wait()
        dbc[ch] = issue_bc(ch)
        if ch + 1 < NCH:
            for c in di.pop(ch + 1):
                c.wait()
            if ch >= 2:
                do.pop(ch - 2).wait()
            da[ch + 1] = issue_a(ch + 1)
        for c in dbc.pop(ch):
            c.wait()
        do[ch] = issue_o(ch)
    for ch in sorted(do):
        do[ch].wait()


import functools


@functools.lru_cache(maxsize=1)
def _make_sc_call():
    return pl.kernel(
        _sc_body,
    out_type=jax.ShapeDtypeStruct((ROWS, WIDTH), jnp.float32),
    mesh=plsc.VectorSubcoreMesh(core_axis_name="c", subcore_axis_name="s"),
    scratch_types=[
        pltpu.VMEM((2, CH), jnp.int32),
        pltpu.VMEM((2, CH), jnp.int32),
        pltpu.VMEM((2, CH), jnp.int32),
        pltpu.VMEM((3, CH, WIDTH), jnp.float32),
        pltpu.VMEM_SHARED((NROWS_PAD, WIDTH), jnp.float32),
        pltpu.SemaphoreType.DMA,
        pltpu.SemaphoreType.DMA,
        pltpu.SemaphoreType.DMA,
        pltpu.SemaphoreType.DMA,
        ],
    )


def _scatter_e():
    e = np.zeros((128, 128), np.float32)
    for j in range(4):
        for bl in range(32):
            e[32 * j + bl, 4 * bl + j] = 1
    return e


_E = _scatter_e()


def kernel(boards, w_quads, w_count):
    batch = boards.shape[0]
    boards3 = boards.reshape(batch // 32, 32, 36)
    w_cat = jnp.concatenate([w_quads, w_count], axis=0)

    table = pl.pallas_call(
        _table_body,
        out_shape=jax.ShapeDtypeStruct((NROWS_PAD, WIDTH), jnp.float32),
    )(jnp.asarray(_M), w_cat)

    rblk = 32
    nr = batch // 32
    shp = jax.ShapeDtypeStruct((nr, 128), jnp.int32)
    bs = pl.BlockSpec((rblk, 128), lambda i: (i, 0))
    ia, ib, ic = pl.pallas_call(
        _idx_body,
        grid=(nr // rblk,),
        in_specs=[
            pl.BlockSpec((rblk, 32, 36), lambda i: (i, 0, 0)),
            pl.BlockSpec((128, 128), lambda i: (0, 0)),
        ],
        out_specs=[bs, bs, bs],
        out_shape=[shp, shp, shp],
    )(boards3, jnp.asarray(_E))

    out = _make_sc_call()(table, ia.reshape(-1), ib.reshape(-1),
                          ic.reshape(-1))
    return out.reshape(batch, 4, WIDTH)


# speedup vs baseline: 1.5487x; 1.5487x over previous
import jax
import jax.numpy as jnp
import numpy as np
from jax import lax
from jax.experimental import pallas as pl
from jax.experimental.pallas import tpu as pltpu
from jax.experimental.pallas import tpu_sc as plsc

BATCH = 16384
WIDTH = 128
NROWS_PAD = 1320
ROWS = BATCH * 4
NC, NS = 2, 16
NW = NC * NS
RPW = ROWS // NW
CH = 128
NCH = RPW // CH

_QSRC = (0, 2, 3, 1)


def _static_m():
    sig = np.zeros((4, 9), dtype=np.int64)
    for p in range(9):
        x, y = divmod(p, 3)
        sig[0, p] = p
        sig[1, p] = 3 * (2 - y) + x
        sig[2, p] = 8 - p
        sig[3, p] = 3 * y + (2 - x)
    inv = np.zeros_like(sig)
    for j in range(4):
        inv[j, sig[j]] = np.arange(9)

    m = np.zeros((NROWS_PAD, 46), np.float32)
    for j in range(4):
        base = 324 * j
        for ia in range(81):
            v = ia
            for t in range(4):
                d = v % 3
                v //= 3
                m[base + ia, 3 * inv[j][t] + d] += 1
        for ib in range(243):
            v = ib
            for t in range(5):
                d = v % 3
                v //= 3
                m[base + 81 + ib, 3 * inv[j][4 + t] + d] += 1
    for c in range(19):
        m[1296 + c, 27 + c] = 1
    return m


_M = _static_m()


def _table_body(m_ref, w_ref, t_ref):
    t_ref[...] = jnp.dot(m_ref[...], w_ref[...],
                         preferred_element_type=jnp.float32)


def _idx_body(b2_ref, gab_ref, gc_ref, oa_ref, ob_ref, oc_ref):
    bf = b2_ref[...].astype(jnp.float32)
    nz = (b2_ref[...] != 0).astype(jnp.float32)
    sab = jnp.dot(bf, gab_ref[...], preferred_element_type=jnp.float32)
    scv = jnp.dot(nz, gc_ref[...],
                  preferred_element_type=jnp.float32).astype(jnp.int32)
    jlane = lax.broadcasted_iota(jnp.int32, oa_ref.shape, 1) % 4
    oa_ref[...] = sab[:, :128].astype(jnp.int32) + 324 * jlane
    ob_ref[...] = sab[:, 128:].astype(jnp.int32) + 324 * jlane + 81
    oc_ref[...] = jnp.where(scv < 19, 1296 + scv, 1315)


def _sc_body(t_hbm, ia_hbm, ib_hbm, ic_hbm, out_hbm,
             ia_v, ib_v, ic_v, acc_v, t_sh,
             sem_i, sem_a, sem_bc, sem_o):
    cid = lax.axis_index("c")
    sid = lax.axis_index("s")
    wid = sid * NC + cid
    base = wid * RPW

    @pl.when(sid == 0)
    def _():
        pltpu.sync_copy(t_hbm, t_sh)

    plsc.subcore_barrier()

    def issue_i(ch):
        r0 = base + ch * CH
        return (
            pltpu.async_copy(ia_hbm.at[pl.ds(r0, CH)], ia_v.at[ch & 1], sem_i),
            pltpu.async_copy(ib_hbm.at[pl.ds(r0, CH)], ib_v.at[ch & 1], sem_i),
            pltpu.async_copy(ic_hbm.at[pl.ds(r0, CH)], ic_v.at[ch & 1], sem_i),
        )

    def issue_a(ch):
        return pltpu.async_copy(t_sh.at[ia_v.at[ch & 1]],
                                acc_v.at[ch % 3], sem_a)

    def issue_bc(ch):
        return (
            pltpu.async_copy(t_sh.at[ib_v.at[ch & 1]],
                             acc_v.at[ch % 3], sem_bc, add=True),
            pltpu.async_copy(t_sh.at[ic_v.at[ch & 1]],
                             acc_v.at[ch % 3], sem_bc, add=True),
        )

    def issue_o(ch):
        return pltpu.async_copy(acc_v.at[ch % 3],
                                out_hbm.at[pl.ds(base + ch * CH, CH)], sem_o)

    di = {0: issue_i(0)}
    for c in di.pop(0):
        c.wait()
    da = {0: issue_a(0)}
    dbc = {}
    do = {}
    for ch in range(NCH):
        if ch + 1 < NCH:
            di[ch + 1] = issue_i(ch + 1)
        da.pop(ch).wait()
        dbc[ch] = issue_bc(ch)
        if ch + 1 < NCH:
            for c in di.pop(ch + 1):
                c.wait()
            if ch >= 2:
                do.pop(ch - 2).wait()
            da[ch + 1] = issue_a(ch + 1)
        for c in dbc.pop(ch):
            c.wait()
        do[ch] = issue_o(ch)
    for ch in sorted(do):
        do[ch].wait()


import functools


@functools.lru_cache(maxsize=1)
def _make_sc_call():
    return pl.kernel(
        _sc_body,
    out_type=jax.ShapeDtypeStruct((ROWS, WIDTH), jnp.float32),
    mesh=plsc.VectorSubcoreMesh(core_axis_name="c", subcore_axis_name="s"),
    scratch_types=[
        pltpu.VMEM((2, CH), jnp.int32),
        pltpu.VMEM((2, CH), jnp.int32),
        pltpu.VMEM((2, CH), jnp.int32),
        pltpu.VMEM((3, CH, WIDTH), jnp.float32),
        pltpu.VMEM_SHARED((NROWS_PAD, WIDTH), jnp.float32),
        pltpu.SemaphoreType.DMA,
        pltpu.SemaphoreType.DMA,
        pltpu.SemaphoreType.DMA,
        pltpu.SemaphoreType.DMA,
        ],
    )


def _scatter_g():
    gab = np.zeros((1152, 256), np.float32)
    gc = np.zeros((1152, 128), np.float32)
    for bl in range(32):
        for j in range(4):
            for t in range(4):
                gab[36 * bl + 9 * _QSRC[j] + t, 4 * bl + j] = 3 ** t
            for t in range(5):
                gab[36 * bl + 9 * _QSRC[j] + 4 + t, 128 + 4 * bl + j] = 3 ** t
            for e in range(36):
                gc[36 * bl + e, 4 * bl + j] = 1
    return gab, gc


_GAB, _GC = _scatter_g()


def kernel(boards, w_quads, w_count):
    batch = boards.shape[0]
    boards2 = boards.reshape(batch // 32, 32 * 36)
    w_cat = jnp.concatenate([w_quads, w_count], axis=0)

    table = pl.pallas_call(
        _table_body,
        out_shape=jax.ShapeDtypeStruct((NROWS_PAD, WIDTH), jnp.float32),
    )(jnp.asarray(_M), w_cat)

    nr = batch // 32
    shp = jax.ShapeDtypeStruct((nr, 128), jnp.int32)
    ia, ib, ic = pl.pallas_call(
        _idx_body,
        out_shape=[shp, shp, shp],
    )(boards2, jnp.asarray(_GAB), jnp.asarray(_GC))

    out = _make_sc_call()(table, ia.reshape(-1), ib.reshape(-1),
                          ic.reshape(-1))
    return out.reshape(batch, 4, WIDTH)


# speedup vs baseline: 1.9380x; 1.2513x over previous
import functools

import jax
import jax.numpy as jnp
import numpy as np
from jax import lax
from jax.experimental import pallas as pl
from jax.experimental.pallas import tpu as pltpu
from jax.experimental.pallas import tpu_sc as plsc

BATCH = 16384
WIDTH = 128
NROWS_PAD = 1320
NC, NS = 2, 16
NW = NC * NS
CH = 128
S_SC = 2048
RPW = S_SC * 4 // NW
NCH = RPW // CH
DBLK = 1024

_QSRC = (0, 2, 3, 1)
_QINV = (0, 3, 1, 2)


def _perm_tables():
    sig = np.zeros((4, 9), dtype=np.int64)
    for p in range(9):
        x, y = divmod(p, 3)
        sig[0, p] = p
        sig[1, p] = 3 * (2 - y) + x
        sig[2, p] = 8 - p
        sig[3, p] = 3 * y + (2 - x)
    inv = np.zeros_like(sig)
    for j in range(4):
        inv[j, sig[j]] = np.arange(9)
    return inv


_INV = _perm_tables()


def _static_m():
    m = np.zeros((NROWS_PAD, 46), np.float32)
    for j in range(4):
        base = 324 * j
        for ia in range(81):
            v = ia
            for t in range(4):
                d = v % 3
                v //= 3
                m[base + ia, 3 * _INV[j][t] + d] += 1
        for ib in range(243):
            v = ib
            for t in range(5):
                d = v % 3
                v //= 3
                m[base + 81 + ib, 3 * _INV[j][4 + t] + d] += 1
    for c in range(19):
        m[1296 + c, 27 + c] = 1
    return m


def _static_mw():
    mw = np.zeros((4, 92, 46), np.float32)
    for i in range(4):
        j = _QINV[i]
        for p in range(9):
            base = 3 * _INV[j][p]
            mw[j, 9 * i + p, base + 1] += 1
            mw[j, 9 * i + p, base] -= 1
            mw[j, 36 + 9 * i + p, base + 2] += 1
            mw[j, 36 + 9 * i + p, base] -= 1
    for j in range(4):
        for c in range(19):
            mw[j, 72 + c, 27 + c] = 1
        for p in range(9):
            mw[j, 91, 3 * p] += 1
    return mw


def _static_g():
    nb = S_SC // 32
    del nb
    gab = np.zeros((1152, 256), np.float32)
    gc = np.zeros((1152, 128), np.float32)
    for bl in range(32):
        for j in range(4):
            for t in range(4):
                gab[36 * bl + 9 * _QSRC[j] + t, 4 * bl + j] = 3 ** t
            for t in range(5):
                gab[36 * bl + 9 * _QSRC[j] + 4 + t, 128 + 4 * bl + j] = 3 ** t
            for e in range(36):
                gc[36 * bl + e, 4 * bl + j] = 1
    return gab, gc


_M = _static_m()
_MW = _static_mw()
_GAB, _GC = _static_g()


def _prep_body(m_ref, mw_ref, w_ref, t_ref, ew_ref):
    w = w_ref[...]
    t_ref[...] = jnp.dot(m_ref[...], w, preferred_element_type=jnp.float32)
    for j in range(4):
        ew_ref[:, 128 * j:128 * (j + 1)] = jnp.dot(
            mw_ref[j], w, preferred_element_type=jnp.float32)


def _idx_body(b2_ref, gab_ref, gc_ref, oa_ref, ob_ref, oc_ref):
    bf = b2_ref[...].astype(jnp.float32)
    nz = (b2_ref[...] != 0).astype(jnp.float32)
    sab = jnp.dot(bf, gab_ref[...], preferred_element_type=jnp.float32)
    scv = jnp.dot(nz, gc_ref[...],
                  preferred_element_type=jnp.float32).astype(jnp.int32)
    jlane = lax.broadcasted_iota(jnp.int32, oa_ref.shape, 1) % 4
    oa_ref[...] = sab[:, :128].astype(jnp.int32) + 324 * jlane
    ob_ref[...] = sab[:, 128:].astype(jnp.int32) + 324 * jlane + 81
    oc_ref[...] = jnp.where(scv < 19, 1296 + scv, 1315)


def _dense_body(b_ref, ew_ref, o_ref):
    blk = b_ref[...]
    x1 = (blk == 1).astype(jnp.float32)
    x2 = (blk == 2).astype(jnp.float32)
    cnt = jnp.sum((blk != 0).astype(jnp.int32), axis=1, keepdims=True)
    oh = (lax.broadcasted_iota(jnp.int32, (blk.shape[0], 19), 1)
          == cnt).astype(jnp.float32)
    one = jnp.ones((blk.shape[0], 1), jnp.float32)
    x = jnp.concatenate([x1, x2, oh, one], axis=1)
    o_ref[...] = jnp.dot(x, ew_ref[...], preferred_element_type=jnp.float32)


def _sc_body(t_hbm, ia_hbm, ib_hbm, ic_hbm, out_hbm,
             ia_v, ib_v, ic_v, acc_v, t_sh,
             sem_i, sem_a, sem_bc, sem_o):
    cid = lax.axis_index("c")
    sid = lax.axis_index("s")
    wid = sid * NC + cid
    base = wid * RPW

    @pl.when(sid == 0)
    def _():
        pltpu.sync_copy(t_hbm, t_sh)

    plsc.subcore_barrier()

    def issue_i(ch):
        r0 = base + ch * CH
        return (
            pltpu.async_copy(ia_hbm.at[pl.ds(r0, CH)], ia_v.at[ch & 1], sem_i),
            pltpu.async_copy(ib_hbm.at[pl.ds(r0, CH)], ib_v.at[ch & 1], sem_i),
            pltpu.async_copy(ic_hbm.at[pl.ds(r0, CH)], ic_v.at[ch & 1], sem_i),
        )

    def issue_a(ch):
        return pltpu.async_copy(t_sh.at[ia_v.at[ch & 1]],
                                acc_v.at[ch % 3], sem_a)

    def issue_bc(ch):
        return (
            pltpu.async_copy(t_sh.at[ib_v.at[ch & 1]],
                             acc_v.at[ch % 3], sem_bc, add=True),
            pltpu.async_copy(t_sh.at[ic_v.at[ch & 1]],
                             acc_v.at[ch % 3], sem_bc, add=True),
        )

    def issue_o(ch):
        return pltpu.async_copy(acc_v.at[ch % 3],
                                out_hbm.at[pl.ds(base + ch * CH, CH)], sem_o)

    di = {0: issue_i(0)}
    for c in di.pop(0):
        c.wait()
    da = {0: issue_a(0)}
    dbc = {}
    do = {}
    for ch in range(NCH):
        if ch + 1 < NCH:
            di[ch + 1] = issue_i(ch + 1)
        da.pop(ch).wait()
        dbc[ch] = issue_bc(ch)
        if ch + 1 < NCH:
            for c in di.pop(ch + 1):
                c.wait()
            if ch >= 2:
                do.pop(ch - 2).wait()
            da[ch + 1] = issue_a(ch + 1)
        for c in dbc.pop(ch):
            c.wait()
        do[ch] = issue_o(ch)
    for ch in sorted(do):
        do[ch].wait()


@functools.lru_cache(maxsize=1)
def _make_sc_call():
    return pl.kernel(
        _sc_body,
        out_type=jax.ShapeDtypeStruct((S_SC * 4, WIDTH), jnp.float32),
        mesh=plsc.VectorSubcoreMesh(core_axis_name="c",
                                    subcore_axis_name="s"),
        scratch_types=[
            pltpu.VMEM((2, CH), jnp.int32),
            pltpu.VMEM((2, CH), jnp.int32),
            pltpu.VMEM((2, CH), jnp.int32),
            pltpu.VMEM((3, CH, WIDTH), jnp.float32),
            pltpu.VMEM_SHARED((NROWS_PAD, WIDTH), jnp.float32),
            pltpu.SemaphoreType.DMA,
            pltpu.SemaphoreType.DMA,
            pltpu.SemaphoreType.DMA,
            pltpu.SemaphoreType.DMA,
        ],
    )


def kernel(boards, w_quads, w_count):
    batch = boards.shape[0]
    bflat = boards.reshape(batch, 36)
    w_cat = jnp.concatenate([w_quads, w_count], axis=0)

    table, ew = pl.pallas_call(
        _prep_body,
        out_shape=[
            jax.ShapeDtypeStruct((NROWS_PAD, WIDTH), jnp.float32),
            jax.ShapeDtypeStruct((92, 512), jnp.float32),
        ],
    )(jnp.asarray(_M), jnp.asarray(_MW), w_cat)

    nrs = S_SC // 32
    boards2s = bflat[:S_SC].reshape(nrs, 32 * 36)
    shp = jax.ShapeDtypeStruct((nrs, 128), jnp.int32)
    ia, ib, ic = pl.pallas_call(
        _idx_body,
        out_shape=[shp, shp, shp],
    )(boards2s, jnp.asarray(_GAB), jnp.asarray(_GC))

    sc_out = _make_sc_call()(table, ia.reshape(-1), ib.reshape(-1),
                             ic.reshape(-1))

    off = S_SC // DBLK
    tc_out = pl.pallas_call(
        _dense_body,
        grid=((batch - S_SC) // DBLK,),
        in_specs=[
            pl.BlockSpec((DBLK, 36), lambda i, o=off: (i + o, 0)),
            pl.BlockSpec((92, 512), lambda i: (0, 0)),
        ],
        out_specs=pl.BlockSpec((DBLK, 512), lambda i, o=off: (i + o, 0)),
        out_shape=jax.ShapeDtypeStruct((batch, 512), jnp.float32),
    )(bflat, ew)

    full = lax.dynamic_update_slice(tc_out, sc_out.reshape(S_SC, 512), (0, 0))
    return full.reshape(batch, 4, WIDTH)


# speedup vs baseline: 2.6547x; 1.3698x over previous
import functools

import jax
import jax.numpy as jnp
import numpy as np
from jax import lax
from jax.experimental import pallas as pl
from jax.experimental.pallas import tpu as pltpu
from jax.experimental.pallas import tpu_sc as plsc

BATCH = 16384
WIDTH = 128
NROWS_PAD = 1320
NC, NS = 2, 16
NW = NC * NS
CH = 128
S_SC = 2048
RPW = S_SC * 4 // NW
NCH = RPW // CH
DBLK = 1024

_QSRC = (0, 2, 3, 1)
_QINV = (0, 3, 1, 2)


def _perm_tables():
    sig = np.zeros((4, 9), dtype=np.int64)
    for p in range(9):
        x, y = divmod(p, 3)
        sig[0, p] = p
        sig[1, p] = 3 * (2 - y) + x
        sig[2, p] = 8 - p
        sig[3, p] = 3 * y + (2 - x)
    inv = np.zeros_like(sig)
    for j in range(4):
        inv[j, sig[j]] = np.arange(9)
    return inv


_INV = _perm_tables()


def _static_m():
    m = np.zeros((NROWS_PAD, 46), np.float32)
    for j in range(4):
        base = 324 * j
        for ia in range(81):
            v = ia
            for t in range(4):
                d = v % 3
                v //= 3
                m[base + ia, 3 * _INV[j][t] + d] += 1
        for ib in range(243):
            v = ib
            for t in range(5):
                d = v % 3
                v //= 3
                m[base + 81 + ib, 3 * _INV[j][4 + t] + d] += 1
    for c in range(19):
        m[1296 + c, 27 + c] = 1
    return m


def _static_mw():
    mw = np.zeros((4, 92, 46), np.float32)
    for i in range(4):
        j = _QINV[i]
        for p in range(9):
            base = 3 * _INV[j][p]
            mw[j, 9 * i + p, base + 1] += 1
            mw[j, 9 * i + p, base] -= 1
            mw[j, 36 + 9 * i + p, base + 2] += 1
            mw[j, 36 + 9 * i + p, base] -= 1
    for j in range(4):
        for c in range(19):
            mw[j, 72 + c, 27 + c] = 1
        for p in range(9):
            mw[j, 91, 3 * p] += 1
    return mw


def _static_g():
    nb = S_SC // 32
    del nb
    gab = np.zeros((1152, 256), np.float32)
    gc = np.zeros((1152, 128), np.float32)
    for bl in range(32):
        for j in range(4):
            for t in range(4):
                gab[36 * bl + 9 * _QSRC[j] + t, 4 * bl + j] = 3 ** t
            for t in range(5):
                gab[36 * bl + 9 * _QSRC[j] + 4 + t, 128 + 4 * bl + j] = 3 ** t
            for e in range(36):
                gc[36 * bl + e, 4 * bl + j] = 1
    return gab, gc


_M = _static_m()
_MW = _static_mw()
_GAB, _GC = _static_g()


def _prep_body(m_ref, mw_ref, w_ref, t_ref, ew_ref):
    w = w_ref[...]
    t_ref[...] = jnp.dot(m_ref[...], w, preferred_element_type=jnp.float32)
    for j in range(4):
        ew_ref[:, 128 * j:128 * (j + 1)] = jnp.dot(
            mw_ref[j], w, preferred_element_type=jnp.float32)


def _idx_body(b2_ref, gab_ref, gc_ref, oa_ref, ob_ref, oc_ref):
    bf = b2_ref[...].astype(jnp.float32)
    nz = (b2_ref[...] != 0).astype(jnp.float32)
    sab = jnp.dot(bf, gab_ref[...], preferred_element_type=jnp.float32)
    scv = jnp.dot(nz, gc_ref[...],
                  preferred_element_type=jnp.float32).astype(jnp.int32)
    jlane = lax.broadcasted_iota(jnp.int32, oa_ref.shape, 1) % 4
    oa_ref[...] = sab[:, :128].astype(jnp.int32) + 324 * jlane
    ob_ref[...] = sab[:, 128:].astype(jnp.int32) + 324 * jlane + 81
    oc_ref[...] = jnp.where(scv < 19, 1296 + scv, 1315)


def _dense_body(b_ref, ew_ref, o_ref):
    blk = b_ref[...]
    x1 = (blk == 1).astype(jnp.float32)
    x2 = (blk == 2).astype(jnp.float32)
    cnt = jnp.sum((blk != 0).astype(jnp.int32), axis=1, keepdims=True)
    oh = (lax.broadcasted_iota(jnp.int32, (blk.shape[0], 19), 1)
          == cnt).astype(jnp.float32)
    one = jnp.ones((blk.shape[0], 1), jnp.float32)
    x = jnp.concatenate([x1, x2, oh, one], axis=1)
    for j in range(4):
        o_ref[:, j, :] = jnp.dot(x, ew_ref[:, 128 * j:128 * (j + 1)],
                                 preferred_element_type=jnp.float32)


def _sc_body(t_hbm, ia_hbm, ib_hbm, ic_hbm, out_hbm,
             ia_v, ib_v, ic_v, acc_v, t_sh,
             sem_i, sem_a, sem_bc, sem_o):
    cid = lax.axis_index("c")
    sid = lax.axis_index("s")
    wid = sid * NC + cid
    base = wid * RPW

    @pl.when(sid == 0)
    def _():
        pltpu.sync_copy(t_hbm, t_sh)

    plsc.subcore_barrier()

    def issue_i(ch):
        r0 = base + ch * CH
        return (
            pltpu.async_copy(ia_hbm.at[pl.ds(r0, CH)], ia_v.at[ch & 1], sem_i),
            pltpu.async_copy(ib_hbm.at[pl.ds(r0, CH)], ib_v.at[ch & 1], sem_i),
            pltpu.async_copy(ic_hbm.at[pl.ds(r0, CH)], ic_v.at[ch & 1], sem_i),
        )

    def issue_a(ch):
        return pltpu.async_copy(t_sh.at[ia_v.at[ch & 1]],
                                acc_v.at[ch % 3], sem_a)

    def issue_bc(ch):
        return (
            pltpu.async_copy(t_sh.at[ib_v.at[ch & 1]],
                             acc_v.at[ch % 3], sem_bc, add=True),
            pltpu.async_copy(t_sh.at[ic_v.at[ch & 1]],
                             acc_v.at[ch % 3], sem_bc, add=True),
        )

    def issue_o(ch):
        return pltpu.async_copy(acc_v.at[ch % 3],
                                out_hbm.at[pl.ds(base + ch * CH, CH)], sem_o)

    di = {0: issue_i(0)}
    for c in di.pop(0):
        c.wait()
    da = {0: issue_a(0)}
    dbc = {}
    do = {}
    for ch in range(NCH):
        if ch + 1 < NCH:
            di[ch + 1] = issue_i(ch + 1)
        da.pop(ch).wait()
        dbc[ch] = issue_bc(ch)
        if ch + 1 < NCH:
            for c in di.pop(ch + 1):
                c.wait()
            if ch >= 2:
                do.pop(ch - 2).wait()
            da[ch + 1] = issue_a(ch + 1)
        for c in dbc.pop(ch):
            c.wait()
        do[ch] = issue_o(ch)
    for ch in sorted(do):
        do[ch].wait()


@functools.lru_cache(maxsize=1)
def _make_sc_call():
    return pl.kernel(
        _sc_body,
        out_type=jax.ShapeDtypeStruct((S_SC * 4, WIDTH), jnp.float32),
        mesh=plsc.VectorSubcoreMesh(core_axis_name="c",
                                    subcore_axis_name="s"),
        scratch_types=[
            pltpu.VMEM((2, CH), jnp.int32),
            pltpu.VMEM((2, CH), jnp.int32),
            pltpu.VMEM((2, CH), jnp.int32),
            pltpu.VMEM((3, CH, WIDTH), jnp.float32),
            pltpu.VMEM_SHARED((NROWS_PAD, WIDTH), jnp.float32),
            pltpu.SemaphoreType.DMA,
            pltpu.SemaphoreType.DMA,
            pltpu.SemaphoreType.DMA,
            pltpu.SemaphoreType.DMA,
        ],
    )


def kernel(boards, w_quads, w_count):
    batch = boards.shape[0]
    bflat = boards.reshape(batch, 36)
    w_cat = jnp.concatenate([w_quads, w_count], axis=0)

    table, ew = pl.pallas_call(
        _prep_body,
        out_shape=[
            jax.ShapeDtypeStruct((NROWS_PAD, WIDTH), jnp.float32),
            jax.ShapeDtypeStruct((92, 512), jnp.float32),
        ],
    )(jnp.asarray(_M), jnp.asarray(_MW), w_cat)

    nrs = S_SC // 32
    boards2s = bflat[:S_SC].reshape(nrs, 32 * 36)
    shp = jax.ShapeDtypeStruct((nrs, 128), jnp.int32)
    ia, ib, ic = pl.pallas_call(
        _idx_body,
        out_shape=[shp, shp, shp],
    )(boards2s, jnp.asarray(_GAB), jnp.asarray(_GC))

    sc_out = _make_sc_call()(table, ia.reshape(-1), ib.reshape(-1),
                             ic.reshape(-1))

    off = S_SC // DBLK
    tc_out = pl.pallas_call(
        _dense_body,
        grid=((batch - S_SC) // DBLK,),
        in_specs=[
            pl.BlockSpec((DBLK, 36), lambda i, o=off: (i + o, 0)),
            pl.BlockSpec((92, 512), lambda i: (0, 0)),
        ],
        out_specs=pl.BlockSpec((DBLK, 4, WIDTH), lambda i, o=off: (i + o, 0, 0)),
        out_shape=jax.ShapeDtypeStruct((batch, 4, WIDTH), jnp.float32),
    )(bflat, ew)

    return lax.dynamic_update_slice(
        tc_out, sc_out.reshape(S_SC, 4, WIDTH), (0, 0, 0))
